# Initial kernel scaffold; baseline (speedup 1.0000x reference)
import jax
import jax.numpy as jnp
from jax.experimental import pallas as pl


def kernel(idx, emb1, emb2, lin1_w, lin1_b, lin2_w, lin2_b):
    raise NotImplementedError("write your pallas kernel here")


# speedup vs baseline: 1.7722x; 1.7722x over previous
import functools

import jax
import jax.numpy as jnp
from jax.experimental import pallas as pl
from jax.experimental.pallas import tpu as pltpu

_ALPHA = 3.0
_TOPK = 20


def _nodevec_kernel(nv1_ref, nv2_ref, w1_ref, b1_ref, w2_ref, b2_ref,
                    lhs_ref, rhs_ref, *, alpha):
    n1 = jnp.tanh(alpha * (
        jnp.dot(nv1_ref[...], w1_ref[...],
                preferred_element_type=jnp.float32) + b1_ref[...]))
    n2 = jnp.tanh(alpha * (
        jnp.dot(nv2_ref[...], w2_ref[...],
                preferred_element_type=jnp.float32) + b2_ref[...]))
    dim = n1.shape[1]
    lhs_ref[:, :dim] = n1
    lhs_ref[:, dim:] = n2
    rhs_ref[:, :dim] = n2
    rhs_ref[:, dim:] = -n1


def _adj_topk_kernel(lhs_ref, rhs_ref, out_ref, *, alpha, topk):
    a = jax.lax.dot_general(lhs_ref[...], rhs_ref[...],
                            (((1,), (1,)), ((), ())),
                            preferred_element_type=jnp.float32)
    adj = jnp.maximum(jnp.tanh(alpha * a), 0.0)

    rt = adj.shape[0]
    keys = jax.lax.bitcast_convert_type(adj, jnp.int32)
    kth = jnp.zeros((rt, 1), jnp.int32)
    for b in range(29, -1, -1):
        cand = kth | (1 << b)
        cnt = jnp.sum(jnp.where(keys >= cand, 1.0, 0.0),
                      axis=-1, keepdims=True)
        kth = jnp.where(cnt >= topk, cand, kth)
    out_ref[...] = jnp.where(keys >= kth, adj, 0.0)


def kernel(idx, emb1, emb2, lin1_w, lin1_b, lin2_w, lin2_b):
    n = int(idx.shape[0])
    nnodes, dim = emb1.shape
    row_tile = 256
    assert n % row_tile == 0

    idx = idx.astype(jnp.int32)
    nv1 = emb1[idx].astype(jnp.float32)
    nv2 = emb2[idx].astype(jnp.float32)
    w1t = lin1_w.T.astype(jnp.float32)
    w2t = lin2_w.T.astype(jnp.float32)
    b1 = lin1_b.reshape(1, dim).astype(jnp.float32)
    b2 = lin2_b.reshape(1, dim).astype(jnp.float32)

    pro_tile = 512
    lhs, rhs = pl.pallas_call(
        functools.partial(_nodevec_kernel, alpha=_ALPHA),
        out_shape=(jax.ShapeDtypeStruct((n, 2 * dim), jnp.float32),
                   jax.ShapeDtypeStruct((n, 2 * dim), jnp.float32)),
        grid=(n // pro_tile,),
        in_specs=[
            pl.BlockSpec((pro_tile, dim), lambda i: (i, 0)),
            pl.BlockSpec((pro_tile, dim), lambda i: (i, 0)),
            pl.BlockSpec((dim, dim), lambda i: (0, 0)),
            pl.BlockSpec((1, dim), lambda i: (0, 0)),
            pl.BlockSpec((dim, dim), lambda i: (0, 0)),
            pl.BlockSpec((1, dim), lambda i: (0, 0)),
        ],
        out_specs=(pl.BlockSpec((pro_tile, 2 * dim), lambda i: (i, 0)),
                   pl.BlockSpec((pro_tile, 2 * dim), lambda i: (i, 0))),
        compiler_params=pltpu.CompilerParams(
            dimension_semantics=("parallel",)),
    )(nv1, nv2, w1t, b1, w2t, b2)

    cost = pl.CostEstimate(
        flops=2 * n * n * (2 * dim),
        transcendentals=n * n,
        bytes_accessed=4 * (n * n + 2 * n * 2 * dim),
    )
    adj = pl.pallas_call(
        functools.partial(_adj_topk_kernel, alpha=_ALPHA, topk=_TOPK),
        out_shape=jax.ShapeDtypeStruct((n, n), jnp.float32),
        grid=(n // row_tile,),
        in_specs=[
            pl.BlockSpec((row_tile, 2 * dim), lambda i: (i, 0)),
            pl.BlockSpec((n, 2 * dim), lambda i: (0, 0)),
        ],
        out_specs=pl.BlockSpec((row_tile, n), lambda i: (i, 0)),
        compiler_params=pltpu.CompilerParams(
            dimension_semantics=("parallel",),
            vmem_limit_bytes=64 * 1024 * 1024,
        ),
        cost_estimate=cost,
    )(lhs, rhs)
    return adj


# speedup vs baseline: 4.8040x; 2.7107x over previous
import functools

import jax
import jax.numpy as jnp
from jax.experimental import pallas as pl
from jax.experimental.pallas import tpu as pltpu

_ALPHA = 3.0
_TOPK = 20


def _count_ge_i16(vals, cand, topk, acc):
    cand16 = cand.astype(jnp.int16)
    w = vals.shape[-1]
    s = None
    for j in range(0, w, 128):
        m = jnp.where(vals[:, j:j + 128] >= cand16,
                      jnp.int16(1), jnp.int16(0))
        s = m if s is None else s + m
    cnt = jnp.sum(s.astype(jnp.int32), axis=-1, keepdims=True)
    return jnp.where(cnt >= topk, cand, acc)


def _topk_mask_exact(adj, topk):
    rt = adj.shape[0]
    keys = jax.lax.bitcast_convert_type(adj, jnp.int32)

    hi = jax.lax.shift_right_logical(keys, 15).astype(jnp.int16)
    pa = jnp.zeros((rt, 1), jnp.int32)
    for b in range(14, -1, -1):
        pa = _count_ge_i16(hi, pa | (1 << b), topk, pa)

    base = pa << 15
    lo = jnp.clip(keys - base, 0, 32767).astype(jnp.int16)
    qb = jnp.zeros((rt, 1), jnp.int32)
    for b in range(14, -1, -1):
        qb = _count_ge_i16(lo, qb | (1 << b), topk, qb)

    kth = base + qb
    return jnp.where(keys >= kth, adj, 0.0)


def _fused_kernel(nv1_ref, nv2_ref, w1_ref, b1_ref, w2_ref, b2_ref,
                  out_ref, nn_ref, *, alpha, topk, dim, row_tile,
                  blocks_per_core):
    c = pl.program_id(0)
    i = pl.program_id(1)

    @pl.when(i == 0)
    def _build_nodevecs():
        nn_ref[:, :dim] = jnp.tanh(alpha * (
            jnp.dot(nv1_ref[...], w1_ref[...],
                    preferred_element_type=jnp.float32) + b1_ref[...]))
        nn_ref[:, dim:] = jnp.tanh(alpha * (
            jnp.dot(nv2_ref[...], w2_ref[...],
                    preferred_element_type=jnp.float32) + b2_ref[...]))

    row0 = (c * blocks_per_core + i) * row_tile
    n1b = nn_ref[pl.ds(row0, row_tile), :dim]
    n2b = nn_ref[pl.ds(row0, row_tile), dim:]
    n1 = nn_ref[:, :dim]
    n2 = nn_ref[:, dim:]
    dn = (((1,), (1,)), ((), ()))
    a = (jax.lax.dot_general(n1b, n2, dn, preferred_element_type=jnp.float32)
         - jax.lax.dot_general(n2b, n1, dn,
                               preferred_element_type=jnp.float32))
    adj = jnp.maximum(jnp.tanh(alpha * a), 0.0)

    cnt1 = jnp.sum(jnp.where(adj >= 1.0, 1.0, 0.0), axis=-1, keepdims=True)
    all_sat = jnp.min(cnt1) >= topk

    @pl.when(all_sat)
    def _fast():
        out_ref[...] = jnp.where(adj >= 1.0, adj, 0.0)

    @pl.when(jnp.logical_not(all_sat))
    def _general():
        out_ref[...] = _topk_mask_exact(adj, topk)


def kernel(idx, emb1, emb2, lin1_w, lin1_b, lin2_w, lin2_b):
    n = int(idx.shape[0])
    nnodes, dim = emb1.shape
    row_tile = 256
    n_cores = 2
    blocks_per_core = n // (row_tile * n_cores)
    assert n == blocks_per_core * row_tile * n_cores

    idx = idx.astype(jnp.int32)
    nv1 = emb1[idx].astype(jnp.float32)
    nv2 = emb2[idx].astype(jnp.float32)
    w1t = lin1_w.T.astype(jnp.float32)
    w2t = lin2_w.T.astype(jnp.float32)
    b1 = lin1_b.reshape(1, dim).astype(jnp.float32)
    b2 = lin2_b.reshape(1, dim).astype(jnp.float32)

    cost = pl.CostEstimate(
        flops=2 * n * n * (2 * dim) + 2 * 2 * n * dim * dim,
        transcendentals=n * n + 2 * n * dim,
        bytes_accessed=4 * (n * n + 4 * n * dim),
    )
    full = lambda c, i: (0, 0)
    adj = pl.pallas_call(
        functools.partial(_fused_kernel, alpha=_ALPHA, topk=_TOPK, dim=dim,
                          row_tile=row_tile, blocks_per_core=blocks_per_core),
        out_shape=jax.ShapeDtypeStruct((n, n), jnp.float32),
        grid=(n_cores, blocks_per_core),
        in_specs=[
            pl.BlockSpec((n, dim), full),
            pl.BlockSpec((n, dim), full),
            pl.BlockSpec((dim, dim), full),
            pl.BlockSpec((1, dim), full),
            pl.BlockSpec((dim, dim), full),
            pl.BlockSpec((1, dim), full),
        ],
        out_specs=pl.BlockSpec(
            (row_tile, n), lambda c, i: (c * blocks_per_core + i, 0)),
        scratch_shapes=[pltpu.VMEM((n, 2 * dim), jnp.float32)],
        compiler_params=pltpu.CompilerParams(
            dimension_semantics=("parallel", "arbitrary"),
            vmem_limit_bytes=60 * 1024 * 1024,
        ),
        cost_estimate=cost,
    )(nv1, nv2, w1t, b1, w2t, b2)
    return adj
